# baseline (speedup 1.0000x reference)
import jax
import jax.numpy as jnp
from jax.experimental import pallas as pl

_VOCAB = 1000
_ROWS = 4096 * 20
_BR = 512
_NB = _ROWS // _BR


def _onehot_body(x_ref, o_ref):
    idx = x_ref[0, 0, :]
    cols = jax.lax.broadcasted_iota(jnp.int32, (_BR, _VOCAB), 1)
    o_ref[...] = (cols == idx[:, None]).astype(jnp.float32)


def kernel(x):
    xf = x.reshape(_NB, 1, _BR).astype(jnp.int32)
    out = pl.pallas_call(
        _onehot_body,
        grid=(_NB,),
        in_specs=[pl.BlockSpec((1, 1, _BR), lambda i: (i, 0, 0))],
        out_specs=pl.BlockSpec((_BR, _VOCAB), lambda i: (i, 0)),
        out_shape=jax.ShapeDtypeStruct((_ROWS, _VOCAB), jnp.float32),
    )(xf)
    return out.reshape(4096, 20, _VOCAB)
